# baseline (speedup 1.0000x reference)
import jax
import jax.numpy as jnp
from jax.experimental import pallas as pl
from jax.experimental.pallas import tpu as pltpu

D = 2048
BATCH = 16384
BM = 512


def _fused_kernel(emb_ref, idx_ref, tt_ref, b_ref, w_ref, out_ref, t_ref):
    @pl.when(pl.program_id(0) == 0)
    def _():
        t_ref[...] = jax.lax.dot_general(
            tt_ref[...], w_ref[:, :D], (((1,), (1,)), ((), ())),
            preferred_element_type=jnp.float32) + b_ref[...]

    acc = jax.lax.dot_general(
        emb_ref[...], w_ref[:, D:], (((1,), (1,)), ((), ())),
        preferred_element_type=jnp.float32)
    idx = idx_ref[...]
    t = t_ref[...]
    addend = jnp.where(
        idx == 0, t[0:1],
        jnp.where(idx == 1, t[1:2],
                  jnp.where(idx == 2, t[2:3], t[3:4])))
    out_ref[...] = acc + addend


def kernel(embedding, task_idxs, task_table, W, b):
    n = W.shape[0]
    idx2d = task_idxs.astype(jnp.int32).reshape(BATCH, 1)
    grid = (BATCH // BM,)
    out = pl.pallas_call(
        _fused_kernel,
        grid=grid,
        in_specs=[
            pl.BlockSpec((BM, D), lambda i: (i, 0)),
            pl.BlockSpec((BM, 1), lambda i: (i, 0)),
            pl.BlockSpec(task_table.shape, lambda i: (0, 0)),
            pl.BlockSpec((1, n), lambda i: (0, 0)),
            pl.BlockSpec((n, 2 * D), lambda i: (0, 0)),
        ],
        out_specs=pl.BlockSpec((BM, n), lambda i: (i, 0)),
        out_shape=jax.ShapeDtypeStruct((BATCH, n), jnp.float32),
        scratch_shapes=[pltpu.VMEM((task_table.shape[0], n), jnp.float32)],
        compiler_params=pltpu.CompilerParams(
            dimension_semantics=("arbitrary",)),
    )(embedding, idx2d, task_table, b.reshape(1, n), W)
    return out


# speedup vs baseline: 1.0049x; 1.0049x over previous
import jax
import jax.numpy as jnp
from jax.experimental import pallas as pl
from jax.experimental.pallas import tpu as pltpu

D = 2048
BATCH = 16384
BM = 512


def _fused_kernel(emb_ref, idx_ref, tt_ref, b_ref, w_ref, out_ref, t_ref):
    i = pl.program_id(0)

    @pl.when(i == 0)
    def _():
        t_ref[...] = jax.lax.dot_general(
            tt_ref[...], w_ref[...], (((1,), (1,)), ((), ())),
            preferred_element_type=jnp.float32) + b_ref[...]

    @pl.when(i > 0)
    def _():
        acc = jax.lax.dot_general(
            emb_ref[...], w_ref[...], (((1,), (1,)), ((), ())),
            preferred_element_type=jnp.float32)
        idx = idx_ref[...]
        t = t_ref[...]
        addend = jnp.where(
            idx == 0, t[0:1],
            jnp.where(idx == 1, t[1:2],
                      jnp.where(idx == 2, t[2:3], t[3:4])))
        out_ref[...] = acc + addend


def kernel(embedding, task_idxs, task_table, W, b):
    n = W.shape[0]
    idx2d = task_idxs.astype(jnp.int32).reshape(BATCH, 1)

    def bi(i):
        return (jnp.maximum(i - 1, 0), 0)

    grid = (BATCH // BM + 1,)
    out = pl.pallas_call(
        _fused_kernel,
        grid=grid,
        in_specs=[
            pl.BlockSpec((BM, D), bi),
            pl.BlockSpec((BM, 1), bi),
            pl.BlockSpec(task_table.shape, lambda i: (0, 0)),
            pl.BlockSpec((1, n), lambda i: (0, 0)),
            pl.BlockSpec((n, D), lambda i: (0, jnp.minimum(i, 1))),
        ],
        out_specs=pl.BlockSpec((BM, n), bi),
        out_shape=jax.ShapeDtypeStruct((BATCH, n), jnp.float32),
        scratch_shapes=[pltpu.VMEM((task_table.shape[0], n), jnp.float32)],
    )(embedding, idx2d, task_table, b.reshape(1, n), W)
    return out


# speedup vs baseline: 1.0219x; 1.0170x over previous
import jax
import jax.numpy as jnp
from jax.experimental import pallas as pl

D = 2048
BATCH = 16384
BM = 1024
KC = 512


def _table_kernel(tt_ref, w1_ref, b_ref, t_ref):
    k = pl.program_id(0)
    part = jax.lax.dot_general(
        tt_ref[...], w1_ref[...], (((1,), (1,)), ((), ())),
        preferred_element_type=jnp.float32)

    @pl.when(k == 0)
    def _():
        t_ref[...] = part + b_ref[...]

    @pl.when(k > 0)
    def _():
        t_ref[...] += part


def _main_kernel(emb_ref, idx_ref, t_ref, w2_ref, out_ref):
    acc = jax.lax.dot_general(
        emb_ref[...], w2_ref[...], (((1,), (1,)), ((), ())),
        preferred_element_type=jnp.float32)
    idx = idx_ref[...]
    t = t_ref[...]
    addend = jnp.where(
        idx == 0, t[0:1],
        jnp.where(idx == 1, t[1:2],
                  jnp.where(idx == 2, t[2:3], t[3:4])))
    out_ref[...] = acc + addend


def kernel(embedding, task_idxs, task_table, W, b):
    n = W.shape[0]
    nt = task_table.shape[0]
    t = pl.pallas_call(
        _table_kernel,
        grid=(D // KC,),
        in_specs=[
            pl.BlockSpec((nt, KC), lambda k: (0, k)),
            pl.BlockSpec((n, KC), lambda k: (0, k)),
            pl.BlockSpec((1, n), lambda k: (0, 0)),
        ],
        out_specs=pl.BlockSpec((nt, n), lambda k: (0, 0)),
        out_shape=jax.ShapeDtypeStruct((nt, n), jnp.float32),
    )(task_table, W, b.reshape(1, n))

    idx2d = task_idxs.astype(jnp.int32).reshape(BATCH, 1)

    grid = (BATCH // BM,)
    out = pl.pallas_call(
        _main_kernel,
        grid=grid,
        in_specs=[
            pl.BlockSpec((BM, D), lambda i: (i, 0)),
            pl.BlockSpec((BM, 1), lambda i: (i, 0)),
            pl.BlockSpec((nt, n), lambda i: (0, 0)),
            pl.BlockSpec((n, D), lambda i: (0, 1)),
        ],
        out_specs=pl.BlockSpec((BM, n), lambda i: (i, 0)),
        out_shape=jax.ShapeDtypeStruct((BATCH, n), jnp.float32),
    )(embedding, idx2d, t, W)
    return out
